# baseline (speedup 1.0000x reference)
import functools

import jax
import jax.numpy as jnp
from jax import lax
from jax.experimental import pallas as pl
from jax.experimental.pallas import tpu as pltpu
from jax.experimental.pallas import tpu_sc as plsc

jax.config.update("jax_enable_x64", True)

N_BATCH = 8
H = 512
W = 512
BH = 36
BW = 36
PER_BATCH = BH * BW
TOTAL = N_BATCH * PER_BATCH
THRESHOLD = 0.5
LANES = 16
NCHUNK = TOTAL // LANES


def _pool_window(x):
    c = x.shape[1]
    x3 = x[0:504].reshape(36, 14, c)
    full = jnp.max(x3, axis=1)
    first = x3[:, 0, :]
    last = x3[:, 13, :]
    zp = jnp.zeros((1, c), jnp.float32)
    prev = jnp.concatenate([zp, last[0:35]], axis=0)
    nxt = jnp.concatenate([first[1:36], x[504:505]], axis=0)
    return jnp.maximum(jnp.maximum(full, prev), nxt)


def _pool_pos_kernel(x_ref, pos_ref, t_ref):
    i = pl.program_id(0)

    @pl.when(i == 0)
    def _init():
        t_ref[0] = 0

    x = x_ref[0, 0]
    rowmax = _pool_window(x)
    pooled_t = _pool_window(rowmax.T)

    act = (pooled_t > THRESHOLD).astype(jnp.float32)

    a_i = lax.broadcasted_iota(jnp.int32, (BH, BW), 0)
    b_i = lax.broadcasted_iota(jnp.int32, (BH, BW), 1)
    lower_incl = (b_i <= a_i).astype(jnp.float32)
    upper_strict = (a_i < b_i).astype(jnp.float32)
    c_col = jnp.dot(lower_incl, act, preferred_element_type=jnp.float32)
    totals = c_col[BH - 1 : BH, :]
    t_excl = jnp.dot(totals, upper_strict, preferred_element_type=jnp.float32)
    c_incl = c_col + t_excl

    n_act = jnp.sum(act)
    t_prev = t_ref[0].astype(jnp.float32)
    suffix = n_act - c_incl
    pos_f = (TOTAL - 1) - t_prev - suffix
    pos = jnp.where(act > 0.0, pos_f, float(TOTAL)).astype(jnp.int32)
    pos_ref[0] = pos
    t_ref[0] = t_ref[0] + n_act.astype(jnp.int32)


@jax.jit
def _tc_positions(mask):
    return pl.pallas_call(
        _pool_pos_kernel,
        grid=(N_BATCH,),
        in_specs=[
            pl.BlockSpec(
                (1, 1, H, W),
                lambda i: (N_BATCH - 1 - i, jnp.int32(0), jnp.int32(0), jnp.int32(0)),
            ),
        ],
        out_specs=pl.BlockSpec(
            (1, BH, BW), lambda i: (N_BATCH - 1 - i, jnp.int32(0), jnp.int32(0))
        ),
        out_shape=jax.ShapeDtypeStruct((N_BATCH, BH, BW), jnp.int32),
        scratch_shapes=[pltpu.SMEM((1,), jnp.int32)],
        compiler_params=pltpu.CompilerParams(
            dimension_semantics=("arbitrary",),
        ),
    )(mask)


@functools.cache
def _sc_scatter_kernel():
    mesh = plsc.VectorSubcoreMesh(core_axis_name="c", subcore_axis_name="s")

    @functools.partial(
        pl.kernel,
        mesh=mesh,
        out_type=(
            jax.ShapeDtypeStruct((TOTAL,), jnp.int32),
            jax.ShapeDtypeStruct((TOTAL,), jnp.int32),
            jax.ShapeDtypeStruct((TOTAL,), jnp.int32),
        ),
        scratch_types=[
            pltpu.VMEM((TOTAL,), jnp.int32),
            pltpu.VMEM((TOTAL,), jnp.int32),
            pltpu.VMEM((TOTAL,), jnp.int32),
        ],
        compiler_params=pltpu.CompilerParams(needs_layout_passes=False),
    )
    def _sc_scatter(pos_hbm, vn_hbm, vbh_hbm, vbw_hbm, out_n, out_bh, out_bw,
                    pos_v, val_v, buf_v):
        cid = lax.axis_index("c")
        sid = lax.axis_index("s")
        wid = lax.mul(sid, jnp.int32(2)) + cid
        lane = lax.iota(jnp.int32, LANES)
        zeros = jnp.zeros((LANES,), jnp.int32)

        for comp, (val_hbm, out_hbm) in enumerate(
            ((vn_hbm, out_n), (vbh_hbm, out_bh), (vbw_hbm, out_bw))
        ):

            @pl.when(wid == jnp.int32(comp))
            def _work(val_hbm=val_hbm, out_hbm=out_hbm):
                pltpu.sync_copy(pos_hbm, pos_v)
                pltpu.sync_copy(val_hbm, val_v)

                def body(k, cnt):
                    base = lax.mul(k, jnp.int32(LANES))
                    idx = pos_v[pl.ds(base, LANES)]
                    vals = val_v[pl.ds(base, LANES)]
                    m = idx < jnp.int32(TOTAL)
                    plsc.store_scatter(buf_v, [idx], vals, mask=m)
                    return cnt + plsc.all_reduce_population_count(m)

                cnt_vec = lax.fori_loop(
                    jnp.int32(0), jnp.int32(NCHUNK), body,
                    jnp.zeros((LANES,), jnp.int32),
                )
                z_vec = jnp.int32(TOTAL) - cnt_vec
                z_scal = jnp.max(z_vec)
                nz_chunks = lax.div(
                    z_scal + jnp.int32(LANES - 1), jnp.int32(LANES)
                )

                def zero_body(k, carry):
                    base = lax.mul(k, jnp.int32(LANES))
                    idxv = base + lane
                    m2 = idxv < z_vec
                    plsc.store_scatter(buf_v, [idxv], zeros, mask=m2)
                    return carry

                lax.fori_loop(jnp.int32(0), nz_chunks, zero_body, jnp.int32(0))

                pltpu.sync_copy(buf_v, out_hbm)

    return _sc_scatter


def kernel(mask):
    pos = _tc_positions(mask)
    f = jnp.arange(TOTAL, dtype=jnp.int32)
    loc = f % PER_BATCH
    vn = f // PER_BATCH
    vbw = loc // BW
    vbh = loc % BW
    n_col, bh_col, bw_col = _sc_scatter_kernel()(
        pos.reshape(TOTAL), vn, vbh, vbw
    )
    return jnp.stack([n_col, bh_col, bw_col], axis=1).astype(jnp.int64)


# speedup vs baseline: 1.0160x; 1.0160x over previous
import functools

import jax
import jax.numpy as jnp
from jax import lax
from jax.experimental import pallas as pl
from jax.experimental.pallas import tpu as pltpu
from jax.experimental.pallas import tpu_sc as plsc

jax.config.update("jax_enable_x64", True)

N_BATCH = 8
H = 512
W = 512
BH = 36
BW = 36
PER_BATCH = BH * BW
TOTAL = N_BATCH * PER_BATCH
THRESHOLD = 0.5
LANES = 16
NCHUNK = TOTAL // LANES


def _pool_window(x):
    c = x.shape[1]
    x3 = x[0:504].reshape(36, 14, c)
    full = jnp.max(x3, axis=1)
    first = x3[:, 0, :]
    last = x3[:, 13, :]
    zp = jnp.zeros((1, c), jnp.float32)
    prev = jnp.concatenate([zp, last[0:35]], axis=0)
    nxt = jnp.concatenate([first[1:36], x[504:505]], axis=0)
    return jnp.maximum(jnp.maximum(full, prev), nxt)


def _pool_pos_kernel(x_ref, pos_ref, t_ref):
    i = pl.program_id(0)

    @pl.when(i == 0)
    def _init():
        t_ref[0] = 0

    x = x_ref[0, 0]
    rowmax = _pool_window(x)
    pooled_t = _pool_window(rowmax.T)

    act = (pooled_t > THRESHOLD).astype(jnp.float32)

    a_i = lax.broadcasted_iota(jnp.int32, (BH, BW), 0)
    b_i = lax.broadcasted_iota(jnp.int32, (BH, BW), 1)
    lower_incl = (b_i <= a_i).astype(jnp.float32)
    upper_strict = (a_i < b_i).astype(jnp.float32)
    c_col = jnp.dot(lower_incl, act, preferred_element_type=jnp.float32)
    totals = c_col[BH - 1 : BH, :]
    t_excl = jnp.dot(totals, upper_strict, preferred_element_type=jnp.float32)
    c_incl = c_col + t_excl

    n_act = jnp.sum(act)
    t_prev = t_ref[0].astype(jnp.float32)
    suffix = n_act - c_incl
    pos_f = (TOTAL - 1) - t_prev - suffix
    pos = jnp.where(act > 0.0, pos_f, float(TOTAL)).astype(jnp.int32)
    pos_ref[0] = pos
    t_ref[0] = t_ref[0] + n_act.astype(jnp.int32)


@jax.jit
def _tc_positions(mask):
    return pl.pallas_call(
        _pool_pos_kernel,
        grid=(N_BATCH,),
        in_specs=[
            pl.BlockSpec(
                (1, 1, H, W),
                lambda i: (N_BATCH - 1 - i, jnp.int32(0), jnp.int32(0), jnp.int32(0)),
            ),
        ],
        out_specs=pl.BlockSpec(
            (1, BH, BW), lambda i: (N_BATCH - 1 - i, jnp.int32(0), jnp.int32(0))
        ),
        out_shape=jax.ShapeDtypeStruct((N_BATCH, BH, BW), jnp.int32),
        scratch_shapes=[pltpu.SMEM((1,), jnp.int32)],
        compiler_params=pltpu.CompilerParams(
            dimension_semantics=("arbitrary",),
        ),
    )(mask)


@functools.cache
def _sc_scatter_kernel():
    mesh = plsc.VectorSubcoreMesh(core_axis_name="c", subcore_axis_name="s")

    @functools.partial(
        pl.kernel,
        mesh=mesh,
        out_type=(
            jax.ShapeDtypeStruct((TOTAL,), jnp.int32),
            jax.ShapeDtypeStruct((TOTAL,), jnp.int32),
            jax.ShapeDtypeStruct((TOTAL,), jnp.int32),
        ),
        scratch_types=[
            pltpu.VMEM((TOTAL,), jnp.int32),
            pltpu.VMEM((TOTAL,), jnp.int32),
            pltpu.VMEM((TOTAL,), jnp.int32),
        ],
        compiler_params=pltpu.CompilerParams(needs_layout_passes=False),
    )
    def _sc_scatter(pos_hbm, vn_hbm, vbh_hbm, vbw_hbm, out_n, out_bh, out_bw,
                    pos_v, val_v, buf_v):
        cid = lax.axis_index("c")
        sid = lax.axis_index("s")
        wid = lax.mul(sid, jnp.int32(2)) + cid
        lane = lax.iota(jnp.int32, LANES)
        zeros = jnp.zeros((LANES,), jnp.int32)

        for comp, (val_hbm, out_hbm) in enumerate(
            ((vn_hbm, out_n), (vbh_hbm, out_bh), (vbw_hbm, out_bw))
        ):

            @pl.when(wid == jnp.int32(comp))
            def _work(val_hbm=val_hbm, out_hbm=out_hbm):
                pltpu.sync_copy(pos_hbm, pos_v)
                pltpu.sync_copy(val_hbm, val_v)

                unroll = 8
                assert NCHUNK % unroll == 0

                def body(k, cnt):
                    base0 = lax.mul(k, jnp.int32(LANES * unroll))
                    for j in range(unroll):
                        base = base0 + jnp.int32(j * LANES)
                        idx = pos_v[pl.ds(base, LANES)]
                        vals = val_v[pl.ds(base, LANES)]
                        m = idx < jnp.int32(TOTAL)
                        plsc.store_scatter(buf_v, [idx], vals, mask=m)
                        cnt = cnt + plsc.all_reduce_population_count(m)
                    return cnt

                cnt_vec = lax.fori_loop(
                    jnp.int32(0), jnp.int32(NCHUNK // unroll), body,
                    jnp.zeros((LANES,), jnp.int32),
                )
                z_vec = jnp.int32(TOTAL) - cnt_vec
                z_scal = jnp.max(z_vec)
                nz_chunks = lax.div(
                    z_scal + jnp.int32(LANES - 1), jnp.int32(LANES)
                )

                def zero_body(k, carry):
                    base = lax.mul(k, jnp.int32(LANES))
                    idxv = base + lane
                    m2 = idxv < z_vec
                    plsc.store_scatter(buf_v, [idxv], zeros, mask=m2)
                    return carry

                lax.fori_loop(jnp.int32(0), nz_chunks, zero_body, jnp.int32(0))

                pltpu.sync_copy(buf_v, out_hbm)

    return _sc_scatter


def kernel(mask):
    pos = _tc_positions(mask)
    f = jnp.arange(TOTAL, dtype=jnp.int32)
    loc = f % PER_BATCH
    vn = f // PER_BATCH
    vbw = loc // BW
    vbh = loc % BW
    n_col, bh_col, bw_col = _sc_scatter_kernel()(
        pos.reshape(TOTAL), vn, vbh, vbw
    )
    return jnp.stack([n_col, bh_col, bw_col], axis=1).astype(jnp.int64)


# speedup vs baseline: 1.0374x; 1.0210x over previous
import functools

import jax
import jax.numpy as jnp
from jax import lax
from jax.experimental import pallas as pl
from jax.experimental.pallas import tpu as pltpu
from jax.experimental.pallas import tpu_sc as plsc

jax.config.update("jax_enable_x64", True)

N_BATCH = 8
H = 512
W = 512
BH = 36
BW = 36
PER_BATCH = BH * BW
TOTAL = N_BATCH * PER_BATCH
THRESHOLD = 0.5
LANES = 16
NCHUNK = TOTAL // LANES


def _pool_window(x):
    c = x.shape[1]
    x3 = x[0:504].reshape(36, 14, c)
    full = jnp.max(x3, axis=1)
    first = x3[:, 0, :]
    last = x3[:, 13, :]
    zp = jnp.zeros((1, c), jnp.float32)
    prev = jnp.concatenate([zp, last[0:35]], axis=0)
    nxt = jnp.concatenate([first[1:36], x[504:505]], axis=0)
    return jnp.maximum(jnp.maximum(full, prev), nxt)


def _pool_pos_kernel(x_ref, pos_ref, cnt_ref, t_ref):
    i = pl.program_id(0)

    @pl.when(i == 0)
    def _init():
        t_ref[0] = 0

    x = x_ref[0, 0]
    rowmax = _pool_window(x)
    pooled_t = _pool_window(rowmax.T)

    act = (pooled_t > THRESHOLD).astype(jnp.float32)

    a_i = lax.broadcasted_iota(jnp.int32, (BH, BW), 0)
    b_i = lax.broadcasted_iota(jnp.int32, (BH, BW), 1)
    lower_incl = (b_i <= a_i).astype(jnp.float32)
    upper_strict = (a_i < b_i).astype(jnp.float32)
    c_col = jnp.dot(lower_incl, act, preferred_element_type=jnp.float32)
    totals = c_col[BH - 1 : BH, :]
    t_excl = jnp.dot(totals, upper_strict, preferred_element_type=jnp.float32)
    c_incl = c_col + t_excl

    n_act = jnp.sum(act)
    t_prev = t_ref[0].astype(jnp.float32)
    suffix = n_act - c_incl
    pos_f = (TOTAL - 1) - t_prev - suffix
    pos = jnp.where(act > 0.0, pos_f, float(TOTAL)).astype(jnp.int32)
    pos_ref[0] = pos
    t_ref[0] = t_ref[0] + n_act.astype(jnp.int32)

    @pl.when(i == N_BATCH - 1)
    def _emit_count():
        cnt_ref[...] = jnp.full((1, 128), t_ref[0], jnp.int32)


@jax.jit
def _tc_positions(mask):
    return pl.pallas_call(
        _pool_pos_kernel,
        grid=(N_BATCH,),
        in_specs=[
            pl.BlockSpec(
                (1, 1, H, W),
                lambda i: (N_BATCH - 1 - i, jnp.int32(0), jnp.int32(0), jnp.int32(0)),
            ),
        ],
        out_specs=[
            pl.BlockSpec(
                (1, BH, BW),
                lambda i: (N_BATCH - 1 - i, jnp.int32(0), jnp.int32(0)),
            ),
            pl.BlockSpec((1, 128), lambda i: (jnp.int32(0), jnp.int32(0))),
        ],
        out_shape=[
            jax.ShapeDtypeStruct((N_BATCH, BH, BW), jnp.int32),
            jax.ShapeDtypeStruct((1, 128), jnp.int32),
        ],
        scratch_shapes=[pltpu.SMEM((1,), jnp.int32)],
        compiler_params=pltpu.CompilerParams(
            dimension_semantics=("arbitrary",),
        ),
    )(mask)


@functools.cache
def _sc_scatter_kernel():
    mesh = plsc.VectorSubcoreMesh(core_axis_name="c", subcore_axis_name="s")

    @functools.partial(
        pl.kernel,
        mesh=mesh,
        out_type=(
            jax.ShapeDtypeStruct((TOTAL,), jnp.int32),
            jax.ShapeDtypeStruct((TOTAL,), jnp.int32),
            jax.ShapeDtypeStruct((TOTAL,), jnp.int32),
        ),
        scratch_types=[
            pltpu.VMEM((TOTAL,), jnp.int32),
            pltpu.VMEM((TOTAL,), jnp.int32),
            pltpu.VMEM((TOTAL,), jnp.int32),
            pltpu.VMEM((LANES,), jnp.int32),
            pltpu.SemaphoreType.DMA,
            pltpu.SemaphoreType.DMA,
        ],
        compiler_params=pltpu.CompilerParams(needs_layout_passes=False),
    )
    def _sc_scatter(pos_hbm, vn_hbm, vbh_hbm, vbw_hbm, cnt_hbm,
                    out_n, out_bh, out_bw,
                    pos_v, val_v, buf_v, cnt_v, sem1, sem2):
        cid = lax.axis_index("c")
        sid = lax.axis_index("s")
        wid = lax.mul(sid, jnp.int32(2)) + cid
        lane = lax.iota(jnp.int32, LANES)
        zeros = jnp.zeros((LANES,), jnp.int32)

        for comp, (val_hbm, out_hbm) in enumerate(
            ((vn_hbm, out_n), (vbh_hbm, out_bh), (vbw_hbm, out_bw))
        ):

            @pl.when(wid == jnp.int32(comp))
            def _work(val_hbm=val_hbm, out_hbm=out_hbm):
                cp1 = pltpu.async_copy(pos_hbm, pos_v, sem1)
                cp2 = pltpu.async_copy(val_hbm, val_v, sem2)
                pltpu.sync_copy(cnt_hbm.at[pl.ds(jnp.int32(0), LANES)], cnt_v)
                cp1.wait()
                cp2.wait()

                unroll = 8
                assert NCHUNK % unroll == 0

                def body(k, carry):
                    base0 = lax.mul(k, jnp.int32(LANES * unroll))
                    for j in range(unroll):
                        base = base0 + jnp.int32(j * LANES)
                        idx = pos_v[pl.ds(base, LANES)]
                        vals = val_v[pl.ds(base, LANES)]
                        m = idx < jnp.int32(TOTAL)
                        plsc.store_scatter(buf_v, [idx], vals, mask=m)
                    return carry

                lax.fori_loop(
                    jnp.int32(0), jnp.int32(NCHUNK // unroll), body,
                    jnp.int32(0),
                )
                z_vec = jnp.int32(TOTAL) - cnt_v[...]
                z_scal = jnp.max(z_vec)
                nz_chunks = lax.div(
                    z_scal + jnp.int32(LANES - 1), jnp.int32(LANES)
                )

                def zero_body(k, carry):
                    base = lax.mul(k, jnp.int32(LANES))
                    idxv = base + lane
                    m2 = idxv < z_vec
                    plsc.store_scatter(buf_v, [idxv], zeros, mask=m2)
                    return carry

                lax.fori_loop(jnp.int32(0), nz_chunks, zero_body, jnp.int32(0))

                pltpu.sync_copy(buf_v, out_hbm)

    return _sc_scatter


def kernel(mask):
    pos, cnt = _tc_positions(mask)
    f = jnp.arange(TOTAL, dtype=jnp.int32)
    loc = f % PER_BATCH
    vn = f // PER_BATCH
    vbw = loc // BW
    vbh = loc % BW
    n_col, bh_col, bw_col = _sc_scatter_kernel()(
        pos.reshape(TOTAL), vn, vbh, vbw, cnt.reshape(128)
    )
    return jnp.stack([n_col, bh_col, bw_col], axis=1).astype(jnp.int64)


# speedup vs baseline: 1.0379x; 1.0005x over previous
import functools

import numpy as np

import jax
import jax.numpy as jnp
from jax import lax
from jax.experimental import pallas as pl
from jax.experimental.pallas import tpu as pltpu
from jax.experimental.pallas import tpu_sc as plsc

jax.config.update("jax_enable_x64", True)

N_BATCH = 8
H = 512
W = 512
BH = 36
BW = 36
PER_BATCH = BH * BW
TOTAL = N_BATCH * PER_BATCH
THRESHOLD = 0.5
LANES = 16
NCHUNK = TOTAL // LANES


def _pool_window(x):
    c = x.shape[1]
    x3 = x[0:504].reshape(36, 14, c)
    full = jnp.max(x3, axis=1)
    first = x3[:, 0, :]
    last = x3[:, 13, :]
    zp = jnp.zeros((1, c), jnp.float32)
    prev = jnp.concatenate([zp, last[0:35]], axis=0)
    nxt = jnp.concatenate([first[1:36], x[504:505]], axis=0)
    return jnp.maximum(jnp.maximum(full, prev), nxt)


def _pool_pos_kernel(x_ref, pos_ref, cnt_ref, t_ref):
    i = pl.program_id(0)

    @pl.when(i == 0)
    def _init():
        t_ref[0] = 0

    x = x_ref[0, 0]
    rowmax = _pool_window(x)
    pooled_t = _pool_window(rowmax.T)

    act = (pooled_t > THRESHOLD).astype(jnp.float32)

    a_i = lax.broadcasted_iota(jnp.int32, (BH, BW), 0)
    b_i = lax.broadcasted_iota(jnp.int32, (BH, BW), 1)
    lower_incl = (b_i <= a_i).astype(jnp.float32)
    upper_strict = (a_i < b_i).astype(jnp.float32)
    c_col = jnp.dot(lower_incl, act, preferred_element_type=jnp.float32)
    totals = c_col[BH - 1 : BH, :]
    t_excl = jnp.dot(totals, upper_strict, preferred_element_type=jnp.float32)
    c_incl = c_col + t_excl

    n_act = jnp.sum(act)
    t_prev = t_ref[0].astype(jnp.float32)
    suffix = n_act - c_incl
    pos_f = (TOTAL - 1) - t_prev - suffix
    pos = jnp.where(act > 0.0, pos_f, float(TOTAL)).astype(jnp.int32)
    pos_ref[0] = pos
    t_ref[0] = t_ref[0] + n_act.astype(jnp.int32)

    @pl.when(i == N_BATCH - 1)
    def _emit_count():
        cnt_ref[...] = jnp.full((1, 128), t_ref[0], jnp.int32)


@jax.jit
def _tc_positions(mask):
    return pl.pallas_call(
        _pool_pos_kernel,
        grid=(N_BATCH,),
        in_specs=[
            pl.BlockSpec(
                (1, 1, H, W),
                lambda i: (N_BATCH - 1 - i, jnp.int32(0), jnp.int32(0), jnp.int32(0)),
            ),
        ],
        out_specs=[
            pl.BlockSpec(
                (1, BH, BW),
                lambda i: (N_BATCH - 1 - i, jnp.int32(0), jnp.int32(0)),
            ),
            pl.BlockSpec((1, 128), lambda i: (jnp.int32(0), jnp.int32(0))),
        ],
        out_shape=[
            jax.ShapeDtypeStruct((N_BATCH, BH, BW), jnp.int32),
            jax.ShapeDtypeStruct((1, 128), jnp.int32),
        ],
        scratch_shapes=[pltpu.SMEM((1,), jnp.int32)],
        compiler_params=pltpu.CompilerParams(
            dimension_semantics=("arbitrary",),
        ),
    )(mask)


@functools.cache
def _sc_scatter_kernel():
    mesh = plsc.VectorSubcoreMesh(core_axis_name="c", subcore_axis_name="s")

    @functools.partial(
        pl.kernel,
        mesh=mesh,
        out_type=(
            jax.ShapeDtypeStruct((TOTAL,), jnp.int32),
            jax.ShapeDtypeStruct((TOTAL,), jnp.int32),
            jax.ShapeDtypeStruct((TOTAL,), jnp.int32),
        ),
        scratch_types=[
            pltpu.VMEM((TOTAL,), jnp.int32),
            pltpu.VMEM((TOTAL,), jnp.int32),
            pltpu.VMEM((TOTAL,), jnp.int32),
            pltpu.VMEM((LANES,), jnp.int32),
            pltpu.SemaphoreType.DMA,
            pltpu.SemaphoreType.DMA,
        ],
        compiler_params=pltpu.CompilerParams(needs_layout_passes=False),
    )
    def _sc_scatter(pos_hbm, vn_hbm, vbh_hbm, vbw_hbm, cnt_hbm,
                    out_n, out_bh, out_bw,
                    pos_v, val_v, buf_v, cnt_v, sem1, sem2):
        cid = lax.axis_index("c")
        sid = lax.axis_index("s")
        wid = lax.mul(sid, jnp.int32(2)) + cid
        lane = lax.iota(jnp.int32, LANES)
        zeros = jnp.zeros((LANES,), jnp.int32)

        for comp, (val_hbm, out_hbm) in enumerate(
            ((vn_hbm, out_n), (vbh_hbm, out_bh), (vbw_hbm, out_bw))
        ):

            @pl.when(wid == jnp.int32(comp))
            def _work(val_hbm=val_hbm, out_hbm=out_hbm):
                cp1 = pltpu.async_copy(pos_hbm, pos_v, sem1)
                cp2 = pltpu.async_copy(val_hbm, val_v, sem2)
                pltpu.sync_copy(cnt_hbm.at[pl.ds(jnp.int32(0), LANES)], cnt_v)
                cp1.wait()
                cp2.wait()

                unroll = 8
                assert NCHUNK % unroll == 0

                def body(k, carry):
                    base0 = lax.mul(k, jnp.int32(LANES * unroll))
                    for j in range(unroll):
                        base = base0 + jnp.int32(j * LANES)
                        idx = pos_v[pl.ds(base, LANES)]
                        vals = val_v[pl.ds(base, LANES)]
                        m = idx < jnp.int32(TOTAL)
                        plsc.store_scatter(buf_v, [idx], vals, mask=m)
                    return carry

                lax.fori_loop(
                    jnp.int32(0), jnp.int32(NCHUNK // unroll), body,
                    jnp.int32(0),
                )
                z_vec = jnp.int32(TOTAL) - cnt_v[...]
                z_scal = jnp.max(z_vec)
                nz_chunks = lax.div(
                    z_scal + jnp.int32(LANES - 1), jnp.int32(LANES)
                )

                def zero_body(k, carry):
                    base = lax.mul(k, jnp.int32(LANES))
                    idxv = base + lane
                    m2 = idxv < z_vec
                    plsc.store_scatter(buf_v, [idxv], zeros, mask=m2)
                    return carry

                lax.fori_loop(jnp.int32(0), nz_chunks, zero_body, jnp.int32(0))

                pltpu.sync_copy(buf_v, out_hbm)

    return _sc_scatter


_F = np.arange(TOTAL, dtype=np.int32)
_VN = np.asarray(_F // PER_BATCH)
_VBW = np.asarray((_F % PER_BATCH) // BW)
_VBH = np.asarray(_F % BW)


def kernel(mask):
    pos, cnt = _tc_positions(mask)
    vn = jnp.asarray(_VN)
    vbh = jnp.asarray(_VBH)
    vbw = jnp.asarray(_VBW)
    n_col, bh_col, bw_col = _sc_scatter_kernel()(
        pos.reshape(TOTAL), vn, vbh, vbw, cnt.reshape(128)
    )
    return jnp.stack([n_col, bh_col, bw_col], axis=1).astype(jnp.int64)


# speedup vs baseline: 1.1441x; 1.1023x over previous
import functools

import numpy as np

import jax
import jax.numpy as jnp
from jax import lax
from jax.experimental import pallas as pl
from jax.experimental.pallas import tpu as pltpu
from jax.experimental.pallas import tpu_sc as plsc

jax.config.update("jax_enable_x64", True)

N_BATCH = 8
H = 512
W = 512
BH = 36
BW = 36
PER_BATCH = BH * BW
TOTAL = N_BATCH * PER_BATCH
THRESHOLD = 0.5
LANES = 16
NCHUNK = TOTAL // LANES


def _pool_window(x):
    c = x.shape[1]
    x3 = x[0:504].reshape(36, 14, c)
    full = jnp.max(x3, axis=1)
    first = x3[:, 0, :]
    last = x3[:, 13, :]
    zp = jnp.zeros((1, c), jnp.float32)
    prev = jnp.concatenate([zp, last[0:35]], axis=0)
    nxt = jnp.concatenate([first[1:36], x[504:505]], axis=0)
    return jnp.maximum(jnp.maximum(full, prev), nxt)


def _pool_pos_kernel(x_ref, pos_ref, cnt_ref, t_ref):
    i = pl.program_id(0)

    @pl.when(i == 0)
    def _init():
        t_ref[0] = 0

    x = x_ref[0, 0]
    rowmax = _pool_window(x)
    pooled_t = _pool_window(rowmax.T)

    act = (pooled_t > THRESHOLD).astype(jnp.float32)

    a_i = lax.broadcasted_iota(jnp.int32, (BH, BW), 0)
    b_i = lax.broadcasted_iota(jnp.int32, (BH, BW), 1)
    lower_incl = (b_i <= a_i).astype(jnp.float32)
    upper_strict = (a_i < b_i).astype(jnp.float32)
    c_col = jnp.dot(lower_incl, act, preferred_element_type=jnp.float32)
    totals = c_col[BH - 1 : BH, :]
    t_excl = jnp.dot(totals, upper_strict, preferred_element_type=jnp.float32)
    c_incl = c_col + t_excl

    n_act = jnp.sum(act)
    t_prev = t_ref[0].astype(jnp.float32)
    suffix = n_act - c_incl
    pos_f = (TOTAL - 1) - t_prev - suffix
    pos = jnp.where(act > 0.0, pos_f, float(TOTAL)).astype(jnp.int32)
    pos_ref[0] = pos
    t_ref[0] = t_ref[0] + n_act.astype(jnp.int32)

    @pl.when(i == N_BATCH - 1)
    def _emit_count():
        cnt_ref[...] = jnp.full((1, 128), t_ref[0], jnp.int32)


@jax.jit
def _tc_positions(mask):
    return pl.pallas_call(
        _pool_pos_kernel,
        grid=(N_BATCH,),
        in_specs=[
            pl.BlockSpec(
                (1, 1, H, W),
                lambda i: (N_BATCH - 1 - i, jnp.int32(0), jnp.int32(0), jnp.int32(0)),
            ),
        ],
        out_specs=[
            pl.BlockSpec(
                (1, BH, BW),
                lambda i: (N_BATCH - 1 - i, jnp.int32(0), jnp.int32(0)),
            ),
            pl.BlockSpec((1, 128), lambda i: (jnp.int32(0), jnp.int32(0))),
        ],
        out_shape=[
            jax.ShapeDtypeStruct((N_BATCH, BH, BW), jnp.int32),
            jax.ShapeDtypeStruct((1, 128), jnp.int32),
        ],
        scratch_shapes=[pltpu.SMEM((1,), jnp.int32)],
        compiler_params=pltpu.CompilerParams(
            dimension_semantics=("arbitrary",),
        ),
    )(mask)


@functools.cache
def _sc_scatter_kernel():
    mesh = plsc.VectorSubcoreMesh(core_axis_name="c", subcore_axis_name="s")

    @functools.partial(
        pl.kernel,
        mesh=mesh,
        out_type=(
            jax.ShapeDtypeStruct((TOTAL,), jnp.int32),
            jax.ShapeDtypeStruct((TOTAL,), jnp.int32),
            jax.ShapeDtypeStruct((TOTAL,), jnp.int32),
        ),
        scratch_types=[
            pltpu.VMEM((TOTAL,), jnp.int32),
            pltpu.VMEM((TOTAL,), jnp.int32),
            pltpu.VMEM((TOTAL,), jnp.int32),
            pltpu.VMEM((LANES,), jnp.int32),
            pltpu.SemaphoreType.DMA,
            pltpu.SemaphoreType.DMA,
        ],
        compiler_params=pltpu.CompilerParams(needs_layout_passes=False),
    )
    def _sc_scatter(pos_hbm, vn_hbm, vbh_hbm, vbw_hbm, cnt_hbm,
                    out_n, out_bh, out_bw,
                    pos_v, val_v, buf_v, cnt_v, sem1, sem2):
        cid = lax.axis_index("c")
        sid = lax.axis_index("s")
        wid = lax.mul(sid, jnp.int32(2)) + cid
        lane = lax.iota(jnp.int32, LANES)
        zeros = jnp.zeros((LANES,), jnp.int32)

        for comp, (val_hbm, out_hbm) in enumerate(
            ((vn_hbm, out_n), (vbh_hbm, out_bh), (vbw_hbm, out_bw))
        ):

            @pl.when(wid == jnp.int32(comp))
            def _work(val_hbm=val_hbm, out_hbm=out_hbm):
                cp1 = pltpu.async_copy(pos_hbm, pos_v, sem1)
                cp2 = pltpu.async_copy(val_hbm, val_v, sem2)
                pltpu.sync_copy(cnt_hbm.at[pl.ds(jnp.int32(0), LANES)], cnt_v)
                cp1.wait()
                cp2.wait()

                @plsc.parallel_loop(
                    jnp.int32(0), jnp.int32(TOTAL), step=jnp.int32(LANES),
                    unroll=8,
                )
                def _scatter_body(base):
                    idx = pos_v[pl.ds(base, LANES)]
                    vals = val_v[pl.ds(base, LANES)]
                    m = idx < jnp.int32(TOTAL)
                    plsc.store_scatter(buf_v, [idx], vals, mask=m)
                z_vec = jnp.int32(TOTAL) - cnt_v[...]
                z_scal = jnp.max(z_vec)
                nz_chunks = lax.div(
                    z_scal + jnp.int32(LANES - 1), jnp.int32(LANES)
                )

                def zero_body(k, carry):
                    base = lax.mul(k, jnp.int32(LANES))
                    idxv = base + lane
                    m2 = idxv < z_vec
                    plsc.store_scatter(buf_v, [idxv], zeros, mask=m2)
                    return carry

                lax.fori_loop(jnp.int32(0), nz_chunks, zero_body, jnp.int32(0))

                pltpu.sync_copy(buf_v, out_hbm)

    return _sc_scatter


_F = np.arange(TOTAL, dtype=np.int32)
_VN = np.asarray(_F // PER_BATCH)
_VBW = np.asarray((_F % PER_BATCH) // BW)
_VBH = np.asarray(_F % BW)


def kernel(mask):
    pos, cnt = _tc_positions(mask)
    vn = jnp.asarray(_VN)
    vbh = jnp.asarray(_VBH)
    vbw = jnp.asarray(_VBW)
    n_col, bh_col, bw_col = _sc_scatter_kernel()(
        pos.reshape(TOTAL), vn, vbh, vbw, cnt.reshape(128)
    )
    return jnp.stack([n_col, bh_col, bw_col], axis=1).astype(jnp.int64)
